# Initial kernel scaffold; baseline (speedup 1.0000x reference)
import jax
import jax.numpy as jnp
from jax.experimental import pallas as pl


def kernel(x, edge_index, W, b):
    raise NotImplementedError("write your pallas kernel here")


# speedup vs baseline: 3.0880x; 3.0880x over previous
import functools

import jax
import jax.numpy as jnp
from jax import lax
from jax.experimental import pallas as pl
from jax.experimental.pallas import tpu as pltpu
from jax.experimental.pallas import tpu_sc as plsc

NC = 2
NS = 16
NW = NC * NS
CHUNK = 128
PAD = 8


def _sc_aggregate(x_aug, ri, ci):
    n = x_aug.shape[0] - CHUNK
    d = x_aug.shape[1]
    e = ri.shape[0]
    assert e % (CHUNK * NW) == 0
    cpw = (e // CHUNK) // NW
    drows = 1000
    nd = n // drows
    assert n % drows == 0 and nd < NS
    nfull = drows // CHUNK
    tail = drows - nfull * CHUNK
    assert tail % 8 == 0

    mesh = plsc.VectorSubcoreMesh(core_axis_name="c", subcore_axis_name="s")

    @functools.partial(
        pl.kernel,
        out_type=jax.ShapeDtypeStruct((NC, n, d), jnp.float32),
        mesh=mesh,
        scratch_types=[
            pltpu.VMEM((CHUNK,), jnp.int32),
            pltpu.VMEM((CHUNK,), jnp.int32),
            pltpu.VMEM((CHUNK, d), jnp.float32),
            pltpu.VMEM_SHARED((n + PAD, d), jnp.float32),
            pltpu.SemaphoreType.DMA,
        ],
    )
    def kernel(x_hbm, ri_hbm, ci_hbm, agg_hbm, idx_r, idx_c, rows, agg_sh,
               sem1):
        cid = lax.axis_index("c")
        sid = lax.axis_index("s")
        wid = sid * NC + cid

        pltpu.sync_copy(x_hbm.at[pl.ds(n, CHUNK)], rows)

        @pl.when(sid < nd)
        def _():
            dbase = sid * drows

            @pl.loop(0, nfull)
            def _(j):
                pltpu.sync_copy(rows,
                                agg_sh.at[pl.ds(dbase + j * CHUNK, CHUNK)])
            o = dbase + nfull * CHUNK
            pltpu.sync_copy(rows.at[pl.ds(0, tail)], agg_sh.at[pl.ds(o, tail)])

        @pl.when(sid == nd)
        def _():
            pltpu.sync_copy(rows.at[pl.ds(0, PAD)], agg_sh.at[pl.ds(n, PAD)])
        plsc.subcore_barrier()

        @pl.loop(0, cpw)
        def _(i):
            off = (wid * cpw + i) * CHUNK
            pltpu.sync_copy(ri_hbm.at[pl.ds(off, CHUNK)], idx_r)
            pltpu.sync_copy(ci_hbm.at[pl.ds(off, CHUNK)], idx_c)
            pltpu.async_copy(x_hbm.at[idx_c], rows, sem1).wait()
            pltpu.async_copy(rows, agg_sh.at[idx_r], sem1, add=True).wait()

        plsc.subcore_barrier()

        @pl.when(sid < nd)
        def _():
            dbase = sid * drows

            @pl.loop(0, nfull)
            def _(j):
                o = dbase + j * CHUNK
                pltpu.sync_copy(agg_sh.at[pl.ds(o, CHUNK)], rows)
                pltpu.sync_copy(rows, agg_hbm.at[cid, pl.ds(o, CHUNK)])
            o = dbase + nfull * CHUNK
            pltpu.sync_copy(agg_sh.at[pl.ds(o, tail)], rows.at[pl.ds(0, tail)])
            pltpu.sync_copy(rows.at[pl.ds(0, tail)],
                            agg_hbm.at[cid, pl.ds(o, tail)])

    return kernel(x_aug, ri, ci)


def _tc_body(x_ref, a_ref, c_ref, wx_ref, wa_ref, b_ref, o_ref):
    xb = x_ref[...]
    agg = a_ref[0] + a_ref[1]
    agg = agg * c_ref[...]
    z = jnp.dot(xb, wx_ref[...], preferred_element_type=jnp.float32)
    z = z + jnp.dot(agg, wa_ref[...], preferred_element_type=jnp.float32)
    z = z + b_ref[...]
    f = jnp.maximum(z, 0.0)
    s = jnp.sum(f * f, axis=1, keepdims=True)
    norm1 = jnp.sqrt(jnp.maximum(s - 1.0, 1e-8))
    g = f / norm1
    zero_mask = jnp.all(g == 0.0, axis=1, keepdims=True)
    g = jnp.where(zero_mask, 1.0, g)
    norm2 = jnp.sqrt(jnp.sum(g * g, axis=1, keepdims=True))
    o_ref[...] = g / jnp.maximum(norm2, 1e-8)


def _tc_finish(x, agg_parts, inv_cnt, wx_t, wa_t, b2):
    n, d = x.shape
    blk = 2000
    assert n % blk == 0
    grid = (n // blk,)
    return pl.pallas_call(
        _tc_body,
        grid=grid,
        in_specs=[
            pl.BlockSpec((blk, d), lambda i: (i, 0)),
            pl.BlockSpec((NC, blk, d), lambda i: (0, i, 0)),
            pl.BlockSpec((blk, d), lambda i: (i, 0)),
            pl.BlockSpec((d, d), lambda i: (0, 0)),
            pl.BlockSpec((d, d), lambda i: (0, 0)),
            pl.BlockSpec((1, d), lambda i: (0, 0)),
        ],
        out_specs=pl.BlockSpec((blk, d), lambda i: (i, 0)),
        out_shape=jax.ShapeDtypeStruct((n, d), jnp.float32),
    )(x, agg_parts, inv_cnt, wx_t, wa_t, b2)


@jax.jit
def kernel(x, edge_index, W, b):
    n, d = x.shape
    e = edge_index.shape[1]
    e_pad = ((e + CHUNK * NW - 1) // (CHUNK * NW)) * (CHUNK * NW)
    npad = e_pad - e
    spread = jnp.arange(npad, dtype=jnp.int32) % PAD
    ri = jnp.concatenate([edge_index[0], n + spread])
    ci = jnp.concatenate([edge_index[1], n + spread])
    x_aug = jnp.concatenate([x, jnp.zeros((CHUNK, d), jnp.float32)], axis=0)
    agg_parts = _sc_aggregate(x_aug, ri, ci)
    cnt = jnp.zeros((n,), jnp.float32).at[edge_index[0]].add(1.0)
    inv_cnt = jnp.broadcast_to((1.0 / jnp.maximum(cnt, 1.0))[:, None], (n, d))
    wx_t = W[:, :d].T
    wa_t = W[:, d:].T
    out = _tc_finish(x, agg_parts, inv_cnt, wx_t, wa_t, b[None, :])
    ones_n = jnp.ones((n, 1), dtype=out.dtype)
    return jnp.concatenate([out, ones_n], axis=1)
